# baseline (speedup 1.0000x reference)
import functools

import jax
import jax.numpy as jnp
from jax import lax
from jax.experimental import pallas as pl
from jax.experimental.pallas import tpu as pltpu
from jax.experimental.pallas import tpu_sc as plsc

_N = 10000
_A = 32
_E = 128
_P625 = 625
_TILE = 1000
_GRID = _N // _TILE

_NWORK = 25
_RPW = _N // _NWORK
_CHUNK = 80
_NCHUNK = _RPW // _CHUNK


def _tc_body(anchor_ref, dt_ref, e_ref, w_ref, b_ref, m1_ref, h_ref, p_scr):
    t = pl.program_id(0)

    @pl.when(t == 0)
    def _init():
        E = e_ref[...]
        W1 = w_ref[:, :_E]
        W2 = w_ref[:, _E:]
        B2 = e_ref[...].reshape(_P625, 16, _E).sum(axis=1)
        r_io = lax.broadcasted_iota(jnp.int32, (_P625, _P625), 0)
        j_io = lax.broadcasted_iota(jnp.int32, (_P625, _P625), 1)
        perm = (jnp.equal((2 * r_io) % _P625, j_io)
                | jnp.equal((2 * r_io + 1) % _P625, j_io)).astype(jnp.float32)
        S = jnp.dot(perm, B2, preferred_element_type=jnp.float32)
        h_ref[...] = lax.dot_general(
            S, W2, (((1,), (1,)), ((), ())),
            preferred_element_type=jnp.float32) * (1.0 / _A)
        rows = lax.broadcasted_iota(jnp.int32, (_A, _N), 1)
        oh = jnp.equal(anchor_ref[...], rows).astype(jnp.float32)
        A32 = jnp.dot(oh, E, preferred_element_type=jnp.float32)
        p_scr[...] = lax.dot_general(
            A32, W1, (((1,), (1,)), ((), ())),
            preferred_element_type=jnp.float32) * (1.0 / _A)

    m1_ref[...] = jnp.dot(dt_ref[...], p_scr[...],
                          preferred_element_type=jnp.float32) + b_ref[...]


def _tc_stage(anchor2d, dists_t, embeds, W, b2d):
    return pl.pallas_call(
        _tc_body,
        grid=(_GRID,),
        in_specs=[
            pl.BlockSpec((_A, 1), lambda t: (0, 0)),
            pl.BlockSpec((_TILE, _A), lambda t: (t, 0)),
            pl.BlockSpec((_N, _E), lambda t: (0, 0)),
            pl.BlockSpec((_E, 2 * _E), lambda t: (0, 0)),
            pl.BlockSpec((1, _E), lambda t: (0, 0)),
        ],
        out_specs=[
            pl.BlockSpec((_TILE, _E), lambda t: (t, 0)),
            pl.BlockSpec((_P625, _E), lambda t: (0, 0)),
        ],
        out_shape=[
            jax.ShapeDtypeStruct((_N, _E), jnp.float32),
            jax.ShapeDtypeStruct((_P625, _E), jnp.float32),
        ],
        scratch_shapes=[pltpu.VMEM((_A, _E), jnp.float32)],
    )(anchor2d, dists_t, embeds, W, b2d)


def _sc_expand_body(m1_hbm, h_hbm, out_hbm, idx_v, hrows_v, m1_v, gsem, msem):
    c = lax.axis_index("c")
    s = lax.axis_index("s")
    wid = s * 2 + c

    @pl.when(wid < _NWORK)
    def _():
        base = wid * _RPW
        cp_m1 = pltpu.async_copy(m1_hbm.at[pl.ds(base, _RPW)], m1_v, msem)
        lane = lax.iota(jnp.int32, 16)
        for j in range(_NCHUNK):
            for v in range(_CHUNK // 16):
                n0 = base + j * _CHUNK + v * 16
                idx_v[j, pl.ds(v * 16, 16)] = (lane + n0) % _P625
        gathers = [
            pltpu.async_copy(h_hbm.at[idx_v.at[j]],
                             hrows_v.at[pl.ds(j * _CHUNK, _CHUNK)], gsem)
            for j in range(_NCHUNK)
        ]
        cp_m1.wait()
        for g in gathers:
            g.wait()

        def body(r, carry):
            for q in range(_E // 16):
                sl = pl.ds(q * 16, 16)
                m1_v[r, sl] = m1_v[r, sl] + hrows_v[r, sl]
            return carry

        lax.fori_loop(0, _RPW, body, 0)
        pltpu.sync_copy(m1_v, out_hbm.at[pl.ds(base, _RPW)])


def _sc_expand(m1, h):
    sc_kernel = functools.partial(
        pl.kernel,
        mesh=plsc.VectorSubcoreMesh(core_axis_name="c", subcore_axis_name="s"),
        out_type=jax.ShapeDtypeStruct((_N, _E), jnp.float32),
        scratch_types=[
            pltpu.VMEM((_NCHUNK, _CHUNK), jnp.int32),
            pltpu.VMEM((_RPW, _E), jnp.float32),
            pltpu.VMEM((_RPW, _E), jnp.float32),
            pltpu.SemaphoreType.DMA,
            pltpu.SemaphoreType.DMA,
        ],
    )(_sc_expand_body)
    return sc_kernel(m1, h)


def kernel(anchor_set_id, dists_array, embeds, W, b):
    anchor2d = anchor_set_id.reshape(_A, 1)
    dists_t = dists_array.T
    b2d = b.reshape(1, _E)
    m1, h = _tc_stage(anchor2d, dists_t, embeds, W, b2d)
    return _sc_expand(m1, h)


# speedup vs baseline: 3.6089x; 3.6089x over previous
import jax
import jax.numpy as jnp
from jax import lax
from jax.experimental import pallas as pl
from jax.experimental.pallas import tpu as pltpu

_N = 10000
_A = 32
_E = 128
_P625 = 625
_TILE = 1000
_GRID = _N // _TILE
_HTAB = _P625 + _TILE


def _tc_body(anchor_ref, d_ref, e_ref, w_ref, b_ref, out_ref,
             p_scr, h_scr, dt_scr):
    t = pl.program_id(0)

    @pl.when(t == 0)
    def _init():
        E = e_ref[...]
        W1 = w_ref[:, :_E]
        W2 = w_ref[:, _E:]
        B2 = e_ref[...].reshape(_P625, 16, _E).sum(axis=1)
        r_io = lax.broadcasted_iota(jnp.int32, (_HTAB, _P625), 0)
        j_io = lax.broadcasted_iota(jnp.int32, (_HTAB, _P625), 1)
        perm = (jnp.equal((2 * r_io) % _P625, j_io)
                | jnp.equal((2 * r_io + 1) % _P625, j_io)).astype(jnp.float32)
        S2 = jnp.dot(perm, B2, preferred_element_type=jnp.float32)
        h_scr[...] = lax.dot_general(
            S2, W2, (((1,), (1,)), ((), ())),
            preferred_element_type=jnp.float32) * (1.0 / _A) + b_ref[...]
        rows = lax.broadcasted_iota(jnp.int32, (_A, _N), 1)
        oh = jnp.equal(anchor_ref[...], rows).astype(jnp.float32)
        A32 = jnp.dot(oh, E, preferred_element_type=jnp.float32)
        p_scr[...] = lax.dot_general(
            A32, W1, (((1,), (1,)), ((), ())),
            preferred_element_type=jnp.float32) * (1.0 / _A)
        dt_scr[...] = jnp.transpose(d_ref[...], (1, 0))

    s_t = (t * _TILE) % _P625
    out_ref[...] = (jnp.dot(dt_scr[pl.ds(t * _TILE, _TILE), :], p_scr[...],
                            preferred_element_type=jnp.float32)
                    + h_scr[pl.ds(s_t, _TILE), :])


def kernel(anchor_set_id, dists_array, embeds, W, b):
    anchor2d = anchor_set_id.reshape(_A, 1)
    b2d = b.reshape(1, _E)
    return pl.pallas_call(
        _tc_body,
        grid=(_GRID,),
        in_specs=[
            pl.BlockSpec((_A, 1), lambda t: (0, 0)),
            pl.BlockSpec((_A, _N), lambda t: (0, 0)),
            pl.BlockSpec((_N, _E), lambda t: (0, 0)),
            pl.BlockSpec((_E, 2 * _E), lambda t: (0, 0)),
            pl.BlockSpec((1, _E), lambda t: (0, 0)),
        ],
        out_specs=pl.BlockSpec((_TILE, _E), lambda t: (t, 0)),
        out_shape=jax.ShapeDtypeStruct((_N, _E), jnp.float32),
        scratch_shapes=[
            pltpu.VMEM((_A, _E), jnp.float32),
            pltpu.VMEM((_HTAB, _E), jnp.float32),
            pltpu.VMEM((_N, _A), jnp.float32),
        ],
    )(anchor2d, dists_array, embeds, W, b2d)
